# baseline (speedup 1.0000x reference)
import functools

import jax
import jax.numpy as jnp
from jax import lax
from jax.experimental import pallas as pl
from jax.experimental.pallas import tpu as pltpu
from jax.experimental.pallas import tpu_sc as plsc

N = 10000
NP = 10240
E = 320000
B = 64
DIN = 128
H = 256
T = 32
L = 6
HALF = 128

NC, NS = 2, 16
K = 128
E_PAD = 327680
EDGES_PER_SUB = E_PAD // NS
NBLK = EDGES_PER_SUB // K
CB = 16
NCHUNK = NBLK // CB
EDGES_PER_W = E_PAD // (NC * NS)
NBLK_DEG = EDGES_PER_W // K
ROWS_PER = NP // NS

R = 512
GRID = NP // R

_mesh = plsc.VectorSubcoreMesh(
    core_axis_name="c", subcore_axis_name="s", num_cores=NC, num_subcores=NS)


@functools.partial(
    pl.kernel,
    out_type=(jax.ShapeDtypeStruct((NP, HALF), jnp.float32),
              jax.ShapeDtypeStruct((NP, HALF), jnp.float32)),
    mesh=_mesh,
    scratch_types=[
        pltpu.VMEM((CB, K), jnp.int32),
        pltpu.VMEM((CB, K), jnp.int32),
        pltpu.VMEM((K, HALF), jnp.float32),
        pltpu.VMEM((K, HALF), jnp.float32),
        pltpu.VMEM_SHARED((NP, HALF), jnp.float32),
        pltpu.SemaphoreType.DMA,
        pltpu.SemaphoreType.DMA,
        pltpu.SemaphoreType.DMA,
        pltpu.SemaphoreType.DMA,
    ],
)
def _seg_sum_sc(hA, hB, src3, dst3, zrows, aggA, aggB,
                sidx2, didx2, rowsA, rowsB, acc, semGA, semGB, semSA, semSB):
    c = lax.axis_index("c")
    s = lax.axis_index("s")
    pltpu.sync_copy(zrows, acc.at[pl.ds(s * ROWS_PER, ROWS_PER)])
    plsc.subcore_barrier()

    def run(h_ref, agg_ref):
        def chunk(ci, carry):
            pltpu.sync_copy(src3.at[s, pl.ds(ci * CB, CB)], sidx2)
            pltpu.sync_copy(dst3.at[s, pl.ds(ci * CB, CB)], didx2)
            pltpu.async_copy(h_ref.at[sidx2.at[0]], rowsA, semGA)
            pltpu.async_copy(h_ref.at[sidx2.at[1]], rowsB, semGB)

            def body(j, carry2):
                i0 = 2 * j
                pltpu.make_async_copy(h_ref.at[sidx2.at[i0]], rowsA,
                                      semGA).wait()
                dA = pltpu.async_copy(rowsA, acc.at[didx2.at[i0]], semSA,
                                      add=True)
                pltpu.make_async_copy(h_ref.at[sidx2.at[i0 + 1]], rowsB,
                                      semGB).wait()
                dB = pltpu.async_copy(rowsB, acc.at[didx2.at[i0 + 1]], semSB,
                                      add=True)
                dA.wait()

                @pl.when(j < CB // 2 - 1)
                def _():
                    pltpu.async_copy(h_ref.at[sidx2.at[i0 + 2]], rowsA, semGA)

                dB.wait()

                @pl.when(j < CB // 2 - 1)
                def _():
                    pltpu.async_copy(h_ref.at[sidx2.at[i0 + 3]], rowsB, semGB)

                return carry2

            lax.fori_loop(0, CB // 2, body, 0)
            return carry

        lax.fori_loop(0, NCHUNK, chunk, 0)
        plsc.subcore_barrier()
        pltpu.sync_copy(acc.at[pl.ds(s * ROWS_PER, ROWS_PER)],
                        agg_ref.at[pl.ds(s * ROWS_PER, ROWS_PER)])

    @pl.when(c == 0)
    def _():
        run(hA, aggA)

    @pl.when(c == 1)
    def _():
        run(hB, aggB)


def _dot(a, b):
    return jnp.dot(a, b, preferred_element_type=jnp.float32,
                   precision=lax.Precision.DEFAULT)


def _proj_body(x_ref, wp_ref, bp_ref, hA_ref, hB_ref):
    h = jnp.maximum(_dot(x_ref[...], wp_ref[...]) + bp_ref[...], 0.0)
    hA_ref[...] = h[:, :HALF]
    hB_ref[...] = h[:, HALF:]


_proj = pl.pallas_call(
    _proj_body,
    grid=(GRID,),
    in_specs=[pl.BlockSpec((R, DIN), lambda i: (i, 0)),
              pl.BlockSpec((DIN, H), lambda i: (0, 0)),
              pl.BlockSpec((1, H), lambda i: (0, 0))],
    out_specs=[pl.BlockSpec((R, HALF), lambda i: (i, 0)),
               pl.BlockSpec((R, HALF), lambda i: (i, 0))],
    out_shape=[jax.ShapeDtypeStruct((NP, HALF), jnp.float32),
               jax.ShapeDtypeStruct((NP, HALF), jnp.float32)],
)


def _layer_body(aA, aB, hA, hB, dg, wl, bl, wr, oA, oB):
    deg = jnp.maximum(dg[...][:, :1], 1.0)
    agg = jnp.concatenate([aA[...], aB[...]], axis=1) / deg
    h = jnp.concatenate([hA[...], hB[...]], axis=1)
    o = jnp.maximum(_dot(agg, wl[...]) + bl[...] + _dot(h, wr[...]), 0.0)
    oA[...] = o[:, :HALF]
    oB[...] = o[:, HALF:]


_layer = pl.pallas_call(
    _layer_body,
    grid=(GRID,),
    in_specs=[pl.BlockSpec((R, HALF), lambda i: (i, 0)),
              pl.BlockSpec((R, HALF), lambda i: (i, 0)),
              pl.BlockSpec((R, HALF), lambda i: (i, 0)),
              pl.BlockSpec((R, HALF), lambda i: (i, 0)),
              pl.BlockSpec((R, HALF), lambda i: (i, 0)),
              pl.BlockSpec((H, H), lambda i: (0, 0)),
              pl.BlockSpec((1, H), lambda i: (0, 0)),
              pl.BlockSpec((H, H), lambda i: (0, 0))],
    out_specs=[pl.BlockSpec((R, HALF), lambda i: (i, 0)),
               pl.BlockSpec((R, HALF), lambda i: (i, 0))],
    out_shape=[jax.ShapeDtypeStruct((NP, HALF), jnp.float32),
               jax.ShapeDtypeStruct((NP, HALF), jnp.float32)],
)


def _pool_body(b3, hA, hB, g_ref, acc, cnt):
    i = pl.program_id(0)

    @pl.when(i == 0)
    def _():
        acc[...] = jnp.zeros_like(acc)
        cnt[...] = jnp.zeros_like(cnt)

    brow = b3[0, 0, :]
    ids = lax.broadcasted_iota(jnp.int32, (B, R), 0)
    mask = (brow[None, :] == ids).astype(jnp.float32)
    h = jnp.concatenate([hA[...], hB[...]], axis=1)
    acc[...] += _dot(mask, h)
    cnt[...] += jnp.sum(mask, axis=1, keepdims=True)

    @pl.when(i == GRID - 1)
    def _():
        g_ref[...] = acc[...] / jnp.maximum(cnt[...][:, :1], 1.0)


_pool = pl.pallas_call(
    _pool_body,
    grid=(GRID,),
    in_specs=[pl.BlockSpec((1, 1, R), lambda i: (i, 0, 0)),
              pl.BlockSpec((R, HALF), lambda i: (i, 0)),
              pl.BlockSpec((R, HALF), lambda i: (i, 0))],
    out_specs=pl.BlockSpec((B, H), lambda i: (0, 0)),
    out_shape=jax.ShapeDtypeStruct((B, H), jnp.float32),
    scratch_shapes=[pltpu.VMEM((B, H), jnp.float32),
                    pltpu.VMEM((B, 128), jnp.float32)],
)


def _head_body(g, ts, wt1, bt1, wt2, bt2, wh1, bh1, wh2, bh2, wh3, bh3,
               wh4, bh4, out):
    t = jnp.maximum(_dot(ts[...], wt1[...]) + bt1[...], 0.0)
    t = _dot(t, wt2[...]) + bt2[...]
    fused = jnp.concatenate([g[...], t], axis=1)
    h1 = jnp.maximum(_dot(fused, wh1[...]) + bh1[...], 0.0)
    h2 = jnp.maximum(_dot(h1, wh2[...]) + bh2[...], 0.0)
    h3 = jnp.maximum(_dot(h2, wh3[...]) + bh3[...], 0.0)
    out[...] = _dot(h3, wh4[...]) + bh4[...]


_head = pl.pallas_call(
    _head_body,
    grid=(1,),
    in_specs=[pl.BlockSpec((B, H), lambda i: (0, 0)),
              pl.BlockSpec((B, T), lambda i: (0, 0)),
              pl.BlockSpec((T, H), lambda i: (0, 0)),
              pl.BlockSpec((1, H), lambda i: (0, 0)),
              pl.BlockSpec((H, H), lambda i: (0, 0)),
              pl.BlockSpec((1, H), lambda i: (0, 0)),
              pl.BlockSpec((2 * H, 512), lambda i: (0, 0)),
              pl.BlockSpec((1, 512), lambda i: (0, 0)),
              pl.BlockSpec((512, 256), lambda i: (0, 0)),
              pl.BlockSpec((1, 256), lambda i: (0, 0)),
              pl.BlockSpec((256, 128), lambda i: (0, 0)),
              pl.BlockSpec((1, 128), lambda i: (0, 0)),
              pl.BlockSpec((128, 1), lambda i: (0, 0)),
              pl.BlockSpec((1, 1), lambda i: (0, 0))],
    out_specs=pl.BlockSpec((B, 1), lambda i: (0, 0)),
    out_shape=jax.ShapeDtypeStruct((B, 1), jnp.float32),
)


def kernel(x, edge_index, batch, transform_seq, Wp, bp, conv_Wl, conv_bl,
           conv_Wr, Wt1, bt1, Wt2, bt2, Wh1, bh1, Wh2, bh2, Wh3, bh3,
           Wh4, bh4):
    src = edge_index[0]
    dst = edge_index[1]
    pad = E_PAD - E
    src_p = jnp.concatenate([src, jnp.zeros((pad,), jnp.int32)])
    dst_p = jnp.concatenate([dst, jnp.full((pad,), N, jnp.int32)])
    perm = jnp.argsort(dst_p)
    src_p = src_p[perm]
    dst_p = dst_p[perm]
    src3 = src_p.reshape(NS, NBLK, K)
    dst3 = dst_p.reshape(NS, NBLK, K)
    zrows = jnp.zeros((ROWS_PER, HALF), jnp.float32)
    ones_np = jnp.ones((NP, HALF), jnp.float32)
    x_p = jnp.concatenate([x, jnp.zeros((NP - N, DIN), jnp.float32)])
    batch3 = jnp.concatenate(
        [batch, jnp.full((NP - N,), B, jnp.int32)]).reshape(GRID, 1, R)

    degf, _ = _seg_sum_sc(ones_np, ones_np, src3, dst3, zrows)
    hA, hB = _proj(x_p, Wp, bp.reshape(1, H))
    hA = hA + 0.0 * degf
    for l in range(L):
        aggA, aggB = _seg_sum_sc(hA, hB, src3, dst3, zrows)
        hA, hB = _layer(aggA, aggB, hA, hB, degf,
                        conv_Wl[l], conv_bl[l].reshape(1, H), conv_Wr[l])
    g = _pool(batch3, hA, hB)
    out = _head(g, transform_seq, Wt1, bt1.reshape(1, H), Wt2,
                bt2.reshape(1, H), Wh1, bh1.reshape(1, 512), Wh2,
                bh2.reshape(1, 256), Wh3, bh3.reshape(1, 128), Wh4,
                bh4.reshape(1, 1))
    return out


# speedup vs baseline: 1.4346x; 1.4346x over previous
import functools

import jax
import jax.numpy as jnp
from jax import lax
from jax.experimental import pallas as pl
from jax.experimental.pallas import tpu as pltpu
from jax.experimental.pallas import tpu_sc as plsc

N = 10000
NP = 10240
E = 320000
B = 64
DIN = 128
H = 256
T = 32
L = 6
HALF = 128

NC, NS = 2, 16
K = 128
E_PAD = 327680
EDGES_PER_SUB = E_PAD // NS
NBLK = EDGES_PER_SUB // K
CB = 16
NCHUNK = NBLK // CB
EDGES_PER_W = E_PAD // (NC * NS)
NBLK_DEG = EDGES_PER_W // K
ROWS_PER = NP // NS
NBLK_D = E_PAD // (NC * NS * K)
NCHUNK_D = NBLK_D // CB

R = 512
GRID = NP // R

_mesh = plsc.VectorSubcoreMesh(
    core_axis_name="c", subcore_axis_name="s", num_cores=NC, num_subcores=NS)


@functools.partial(
    pl.kernel,
    out_type=(jax.ShapeDtypeStruct((NP, HALF), jnp.float32),
              jax.ShapeDtypeStruct((NP, HALF), jnp.float32)),
    mesh=_mesh,
    scratch_types=[
        pltpu.VMEM((CB, K), jnp.int32),
        pltpu.VMEM((CB, K), jnp.int32),
        pltpu.VMEM((K, HALF), jnp.float32),
        pltpu.VMEM((K, HALF), jnp.float32),
        pltpu.VMEM_SHARED((NP, HALF), jnp.float32),
        pltpu.SemaphoreType.DMA,
        pltpu.SemaphoreType.DMA,
        pltpu.SemaphoreType.DMA,
        pltpu.SemaphoreType.DMA,
    ],
)
def _seg_sum_sc(hA, hB, src3, dst3, zrows, aggA, aggB,
                sidx2, didx2, rowsA, rowsB, acc, semGA, semGB, semSA, semSB):
    c = lax.axis_index("c")
    s = lax.axis_index("s")
    pltpu.sync_copy(zrows, acc.at[pl.ds(s * ROWS_PER, ROWS_PER)])
    plsc.subcore_barrier()

    def run(h_ref, agg_ref):
        def chunk(ci, carry):
            pltpu.sync_copy(src3.at[s, pl.ds(ci * CB, CB)], sidx2)
            pltpu.sync_copy(dst3.at[s, pl.ds(ci * CB, CB)], didx2)
            pltpu.async_copy(h_ref.at[sidx2.at[0]], rowsA, semGA)
            pltpu.async_copy(h_ref.at[sidx2.at[1]], rowsB, semGB)

            def body(j, carry2):
                i0 = 2 * j
                pltpu.make_async_copy(h_ref.at[sidx2.at[i0]], rowsA,
                                      semGA).wait()
                dA = pltpu.async_copy(rowsA, acc.at[didx2.at[i0]], semSA,
                                      add=True)
                pltpu.make_async_copy(h_ref.at[sidx2.at[i0 + 1]], rowsB,
                                      semGB).wait()
                dB = pltpu.async_copy(rowsB, acc.at[didx2.at[i0 + 1]], semSB,
                                      add=True)
                dA.wait()

                @pl.when(j < CB // 2 - 1)
                def _():
                    pltpu.async_copy(h_ref.at[sidx2.at[i0 + 2]], rowsA, semGA)

                dB.wait()

                @pl.when(j < CB // 2 - 1)
                def _():
                    pltpu.async_copy(h_ref.at[sidx2.at[i0 + 3]], rowsB, semGB)

                return carry2

            lax.fori_loop(0, CB // 2, body, 0)
            return carry

        lax.fori_loop(0, NCHUNK, chunk, 0)
        plsc.subcore_barrier()
        pltpu.sync_copy(acc.at[pl.ds(s * ROWS_PER, ROWS_PER)],
                        agg_ref.at[pl.ds(s * ROWS_PER, ROWS_PER)])

    @pl.when(c == 0)
    def _():
        run(hA, aggA)

    @pl.when(c == 1)
    def _():
        run(hB, aggB)


@functools.partial(
    pl.kernel,
    out_type=(jax.ShapeDtypeStruct((NP, HALF), jnp.float32),
              jax.ShapeDtypeStruct((NP, HALF), jnp.float32)),
    mesh=_mesh,
    scratch_types=[
        pltpu.VMEM((CB, K), jnp.int32),
        pltpu.VMEM((K, HALF), jnp.float32),
        pltpu.VMEM_SHARED((NP, HALF), jnp.float32),
    ],
)
def _deg_sc(dstd, ones_hbm, zrows, deg0, deg1, didx2, ones_v, acc):
    c = lax.axis_index("c")
    s = lax.axis_index("s")
    w = c * NS + s
    pltpu.sync_copy(zrows, acc.at[pl.ds(s * ROWS_PER, ROWS_PER)])
    pltpu.sync_copy(ones_hbm, ones_v)
    plsc.subcore_barrier()

    def chunk(ci, carry):
        pltpu.sync_copy(dstd.at[w, pl.ds(ci * CB, CB)], didx2)

        def body(j, carry2):
            pltpu.sync_copy(ones_v, acc.at[didx2.at[j]], add=True)
            return carry2

        lax.fori_loop(0, CB, body, 0)
        return carry

    lax.fori_loop(0, NCHUNK_D, chunk, 0)
    plsc.subcore_barrier()

    @pl.when(c == 0)
    def _():
        pltpu.sync_copy(acc.at[pl.ds(s * ROWS_PER, ROWS_PER)],
                        deg0.at[pl.ds(s * ROWS_PER, ROWS_PER)])

    @pl.when(c == 1)
    def _():
        pltpu.sync_copy(acc.at[pl.ds(s * ROWS_PER, ROWS_PER)],
                        deg1.at[pl.ds(s * ROWS_PER, ROWS_PER)])


def _dot(a, b):
    return jnp.dot(a, b, preferred_element_type=jnp.float32,
                   precision=lax.Precision.DEFAULT)


def _proj_body(x_ref, wp_ref, bp_ref, hA_ref, hB_ref):
    h = jnp.maximum(_dot(x_ref[...], wp_ref[...]) + bp_ref[...], 0.0)
    hA_ref[...] = h[:, :HALF]
    hB_ref[...] = h[:, HALF:]


_proj = pl.pallas_call(
    _proj_body,
    grid=(GRID,),
    in_specs=[pl.BlockSpec((R, DIN), lambda i: (i, 0)),
              pl.BlockSpec((DIN, H), lambda i: (0, 0)),
              pl.BlockSpec((1, H), lambda i: (0, 0))],
    out_specs=[pl.BlockSpec((R, HALF), lambda i: (i, 0)),
               pl.BlockSpec((R, HALF), lambda i: (i, 0))],
    out_shape=[jax.ShapeDtypeStruct((NP, HALF), jnp.float32),
               jax.ShapeDtypeStruct((NP, HALF), jnp.float32)],
)


def _linear_body(hA, hB, wr, bl, oA, oB):
    h = jnp.concatenate([hA[...], hB[...]], axis=1)
    o = _dot(h, wr[...]) + bl[...]
    oA[...] = o[:, :HALF]
    oB[...] = o[:, HALF:]


_linear = pl.pallas_call(
    _linear_body,
    grid=(GRID,),
    in_specs=[pl.BlockSpec((R, HALF), lambda i: (i, 0)),
              pl.BlockSpec((R, HALF), lambda i: (i, 0)),
              pl.BlockSpec((H, H), lambda i: (0, 0)),
              pl.BlockSpec((1, H), lambda i: (0, 0))],
    out_specs=[pl.BlockSpec((R, HALF), lambda i: (i, 0)),
               pl.BlockSpec((R, HALF), lambda i: (i, 0))],
    out_shape=[jax.ShapeDtypeStruct((NP, HALF), jnp.float32),
               jax.ShapeDtypeStruct((NP, HALF), jnp.float32)],
)


def _combine_body(aA, aB, d0, d1, rA, rB, wl, oA, oB):
    deg = jnp.maximum(d0[...][:, :1] + d1[...][:, :1], 1.0)
    agg = jnp.concatenate([aA[...], aB[...]], axis=1) / deg
    hr = jnp.concatenate([rA[...], rB[...]], axis=1)
    o = jnp.maximum(_dot(agg, wl[...]) + hr, 0.0)
    oA[...] = o[:, :HALF]
    oB[...] = o[:, HALF:]


_combine = pl.pallas_call(
    _combine_body,
    grid=(GRID,),
    in_specs=[pl.BlockSpec((R, HALF), lambda i: (i, 0)),
              pl.BlockSpec((R, HALF), lambda i: (i, 0)),
              pl.BlockSpec((R, HALF), lambda i: (i, 0)),
              pl.BlockSpec((R, HALF), lambda i: (i, 0)),
              pl.BlockSpec((R, HALF), lambda i: (i, 0)),
              pl.BlockSpec((R, HALF), lambda i: (i, 0)),
              pl.BlockSpec((H, H), lambda i: (0, 0))],
    out_specs=[pl.BlockSpec((R, HALF), lambda i: (i, 0)),
               pl.BlockSpec((R, HALF), lambda i: (i, 0))],
    out_shape=[jax.ShapeDtypeStruct((NP, HALF), jnp.float32),
               jax.ShapeDtypeStruct((NP, HALF), jnp.float32)],
)


def _pool_body(b3, hA, hB, g_ref, acc, cnt):
    i = pl.program_id(0)

    @pl.when(i == 0)
    def _():
        acc[...] = jnp.zeros_like(acc)
        cnt[...] = jnp.zeros_like(cnt)

    brow = b3[0, 0, :]
    ids = lax.broadcasted_iota(jnp.int32, (B, R), 0)
    mask = (brow[None, :] == ids).astype(jnp.float32)
    h = jnp.concatenate([hA[...], hB[...]], axis=1)
    acc[...] += _dot(mask, h)
    cnt[...] += jnp.sum(mask, axis=1, keepdims=True)

    @pl.when(i == GRID - 1)
    def _():
        g_ref[...] = acc[...] / jnp.maximum(cnt[...][:, :1], 1.0)


_pool = pl.pallas_call(
    _pool_body,
    grid=(GRID,),
    in_specs=[pl.BlockSpec((1, 1, R), lambda i: (i, 0, 0)),
              pl.BlockSpec((R, HALF), lambda i: (i, 0)),
              pl.BlockSpec((R, HALF), lambda i: (i, 0))],
    out_specs=pl.BlockSpec((B, H), lambda i: (0, 0)),
    out_shape=jax.ShapeDtypeStruct((B, H), jnp.float32),
    scratch_shapes=[pltpu.VMEM((B, H), jnp.float32),
                    pltpu.VMEM((B, 128), jnp.float32)],
)


def _head_body(g, ts, wt1, bt1, wt2, bt2, wh1, bh1, wh2, bh2, wh3, bh3,
               wh4, bh4, out):
    t = jnp.maximum(_dot(ts[...], wt1[...]) + bt1[...], 0.0)
    t = _dot(t, wt2[...]) + bt2[...]
    fused = jnp.concatenate([g[...], t], axis=1)
    h1 = jnp.maximum(_dot(fused, wh1[...]) + bh1[...], 0.0)
    h2 = jnp.maximum(_dot(h1, wh2[...]) + bh2[...], 0.0)
    h3 = jnp.maximum(_dot(h2, wh3[...]) + bh3[...], 0.0)
    out[...] = _dot(h3, wh4[...]) + bh4[...]


_head = pl.pallas_call(
    _head_body,
    grid=(1,),
    in_specs=[pl.BlockSpec((B, H), lambda i: (0, 0)),
              pl.BlockSpec((B, T), lambda i: (0, 0)),
              pl.BlockSpec((T, H), lambda i: (0, 0)),
              pl.BlockSpec((1, H), lambda i: (0, 0)),
              pl.BlockSpec((H, H), lambda i: (0, 0)),
              pl.BlockSpec((1, H), lambda i: (0, 0)),
              pl.BlockSpec((2 * H, 512), lambda i: (0, 0)),
              pl.BlockSpec((1, 512), lambda i: (0, 0)),
              pl.BlockSpec((512, 256), lambda i: (0, 0)),
              pl.BlockSpec((1, 256), lambda i: (0, 0)),
              pl.BlockSpec((256, 128), lambda i: (0, 0)),
              pl.BlockSpec((1, 128), lambda i: (0, 0)),
              pl.BlockSpec((128, 1), lambda i: (0, 0)),
              pl.BlockSpec((1, 1), lambda i: (0, 0))],
    out_specs=pl.BlockSpec((B, 1), lambda i: (0, 0)),
    out_shape=jax.ShapeDtypeStruct((B, 1), jnp.float32),
)


def kernel(x, edge_index, batch, transform_seq, Wp, bp, conv_Wl, conv_bl,
           conv_Wr, Wt1, bt1, Wt2, bt2, Wh1, bh1, Wh2, bh2, Wh3, bh3,
           Wh4, bh4):
    src = edge_index[0]
    dst = edge_index[1]
    pad = E_PAD - E
    src_p = jnp.concatenate([src, jnp.zeros((pad,), jnp.int32)])
    dst_p = jnp.concatenate([dst, jnp.full((pad,), N, jnp.int32)])
    src3 = src_p.reshape(NS, NBLK, K)
    dst3 = dst_p.reshape(NS, NBLK, K)
    zrows = jnp.zeros((ROWS_PER, HALF), jnp.float32)
    onesk = jnp.ones((K, HALF), jnp.float32)
    dstd = dst_p.reshape(NC * NS, NBLK_D, K)
    x_p = jnp.concatenate([x, jnp.zeros((NP - N, DIN), jnp.float32)])
    batch3 = jnp.concatenate(
        [batch, jnp.full((NP - N,), B, jnp.int32)]).reshape(GRID, 1, R)

    deg0, deg1 = _deg_sc(dstd, onesk, zrows)
    hA, hB = _proj(x_p, Wp, bp.reshape(1, H))
    hA = hA + 0.0 * deg0
    for l in range(L):
        hrA, hrB = _linear(hA, hB, conv_Wr[l], conv_bl[l].reshape(1, H))
        aggA, aggB = _seg_sum_sc(hA, hB, src3, dst3, zrows)
        hA, hB = _combine(aggA, aggB, deg0, deg1, hrA, hrB, conv_Wl[l])
    g = _pool(batch3, hA, hB)
    out = _head(g, transform_seq, Wt1, bt1.reshape(1, H), Wt2,
                bt2.reshape(1, H), Wh1, bh1.reshape(1, 512), Wh2,
                bh2.reshape(1, 256), Wh3, bh3.reshape(1, 128), Wh4,
                bh4.reshape(1, 1))
    return out


# speedup vs baseline: 1.4370x; 1.0016x over previous
import functools

import jax
import jax.numpy as jnp
from jax import lax
from jax.experimental import pallas as pl
from jax.experimental.pallas import tpu as pltpu
from jax.experimental.pallas import tpu_sc as plsc

N = 10000
NP = 10240
E = 320000
B = 64
DIN = 128
H = 256
T = 32
L = 6
HALF = 128

NC, NS = 2, 16
K = 128
E_PAD = 327680
EDGES_PER_SUB = E_PAD // NS
NBLK = EDGES_PER_SUB // K
CB = 16
NCHUNK = NBLK // CB
ROWS_PER = NP // NS
NBLK_D = E_PAD // (NC * NS * K)
NCHUNK_D = NBLK_D // CB

R = 512
GRID = NP // R

_mesh = plsc.VectorSubcoreMesh(
    core_axis_name="c", subcore_axis_name="s", num_cores=NC, num_subcores=NS)


@functools.partial(
    pl.kernel,
    out_type=(jax.ShapeDtypeStruct((NP, HALF), jnp.float32),
              jax.ShapeDtypeStruct((NP, HALF), jnp.float32)),
    mesh=_mesh,
    scratch_types=[
        pltpu.VMEM((CB, K), jnp.int32),
        pltpu.VMEM((CB, K), jnp.int32),
        pltpu.VMEM((K, HALF), jnp.float32),
        pltpu.VMEM((K, HALF), jnp.float32),
        pltpu.VMEM_SHARED((NP, HALF), jnp.float32),
        pltpu.SemaphoreType.DMA,
        pltpu.SemaphoreType.DMA,
        pltpu.SemaphoreType.DMA,
        pltpu.SemaphoreType.DMA,
    ],
)
def _seg_sum_sc(hA, hB, src3, dst3, zrows, aggA, aggB,
                sidx2, didx2, rowsA, rowsB, acc, semGA, semGB, semSA, semSB):
    c = lax.axis_index("c")
    s = lax.axis_index("s")
    pltpu.sync_copy(zrows, acc.at[pl.ds(s * ROWS_PER, ROWS_PER)])
    plsc.subcore_barrier()

    def run(h_ref, agg_ref):
        def chunk(ci, carry):
            pltpu.sync_copy(src3.at[s, pl.ds(ci * CB, CB)], sidx2)
            pltpu.sync_copy(dst3.at[s, pl.ds(ci * CB, CB)], didx2)
            pltpu.async_copy(h_ref.at[sidx2.at[0]], rowsA, semGA)
            pltpu.async_copy(h_ref.at[sidx2.at[1]], rowsB, semGB)

            def body(j, carry2):
                i0 = 2 * j
                pltpu.make_async_copy(h_ref.at[sidx2.at[i0]], rowsA,
                                      semGA).wait()
                dA = pltpu.async_copy(rowsA, acc.at[didx2.at[i0]], semSA,
                                      add=True)
                pltpu.make_async_copy(h_ref.at[sidx2.at[i0 + 1]], rowsB,
                                      semGB).wait()
                dB = pltpu.async_copy(rowsB, acc.at[didx2.at[i0 + 1]], semSB,
                                      add=True)
                dA.wait()

                @pl.when(j < CB // 2 - 1)
                def _():
                    pltpu.async_copy(h_ref.at[sidx2.at[i0 + 2]], rowsA, semGA)

                dB.wait()

                @pl.when(j < CB // 2 - 1)
                def _():
                    pltpu.async_copy(h_ref.at[sidx2.at[i0 + 3]], rowsB, semGB)

                return carry2

            lax.fori_loop(0, CB // 2, body, 0)
            return carry

        lax.fori_loop(0, NCHUNK, chunk, 0)
        plsc.subcore_barrier()
        pltpu.sync_copy(acc.at[pl.ds(s * ROWS_PER, ROWS_PER)],
                        agg_ref.at[pl.ds(s * ROWS_PER, ROWS_PER)])

    @pl.when(c == 0)
    def _():
        run(hA, aggA)

    @pl.when(c == 1)
    def _():
        run(hB, aggB)


@functools.partial(
    pl.kernel,
    out_type=(jax.ShapeDtypeStruct((NP, HALF), jnp.float32),
              jax.ShapeDtypeStruct((NP, HALF), jnp.float32)),
    mesh=_mesh,
    scratch_types=[
        pltpu.VMEM((CB, K), jnp.int32),
        pltpu.VMEM((K, HALF), jnp.float32),
        pltpu.VMEM_SHARED((NP, HALF), jnp.float32),
    ],
)
def _deg_sc(dstd, ones_hbm, zrows, deg0, deg1, didx2, ones_v, acc):
    c = lax.axis_index("c")
    s = lax.axis_index("s")
    w = c * NS + s
    pltpu.sync_copy(zrows, acc.at[pl.ds(s * ROWS_PER, ROWS_PER)])
    pltpu.sync_copy(ones_hbm, ones_v)
    plsc.subcore_barrier()

    def chunk(ci, carry):
        pltpu.sync_copy(dstd.at[w, pl.ds(ci * CB, CB)], didx2)

        def body(j, carry2):
            pltpu.sync_copy(ones_v, acc.at[didx2.at[j]], add=True)
            return carry2

        lax.fori_loop(0, CB, body, 0)
        return carry

    lax.fori_loop(0, NCHUNK_D, chunk, 0)
    plsc.subcore_barrier()

    @pl.when(c == 0)
    def _():
        pltpu.sync_copy(acc.at[pl.ds(s * ROWS_PER, ROWS_PER)],
                        deg0.at[pl.ds(s * ROWS_PER, ROWS_PER)])

    @pl.when(c == 1)
    def _():
        pltpu.sync_copy(acc.at[pl.ds(s * ROWS_PER, ROWS_PER)],
                        deg1.at[pl.ds(s * ROWS_PER, ROWS_PER)])


def _dot(a, b):
    return jnp.dot(a, b, preferred_element_type=jnp.float32,
                   precision=lax.Precision.DEFAULT)


def _proj_body(x_ref, wp_ref, bp_ref, hA_ref, hB_ref):
    h = jnp.maximum(_dot(x_ref[...], wp_ref[...]) + bp_ref[...], 0.0)
    hA_ref[...] = h[:, :HALF]
    hB_ref[...] = h[:, HALF:]


_proj = pl.pallas_call(
    _proj_body,
    grid=(GRID,),
    in_specs=[pl.BlockSpec((R, DIN), lambda i: (i, 0)),
              pl.BlockSpec((DIN, H), lambda i: (0, 0)),
              pl.BlockSpec((1, H), lambda i: (0, 0))],
    out_specs=[pl.BlockSpec((R, HALF), lambda i: (i, 0)),
               pl.BlockSpec((R, HALF), lambda i: (i, 0))],
    out_shape=[jax.ShapeDtypeStruct((NP, HALF), jnp.float32),
               jax.ShapeDtypeStruct((NP, HALF), jnp.float32)],
)


def _linear_body(hA, hB, wr, bl, oA, oB):
    h = jnp.concatenate([hA[...], hB[...]], axis=1)
    o = _dot(h, wr[...]) + bl[...]
    oA[...] = o[:, :HALF]
    oB[...] = o[:, HALF:]


_linear = pl.pallas_call(
    _linear_body,
    grid=(GRID,),
    in_specs=[pl.BlockSpec((R, HALF), lambda i: (i, 0)),
              pl.BlockSpec((R, HALF), lambda i: (i, 0)),
              pl.BlockSpec((H, H), lambda i: (0, 0)),
              pl.BlockSpec((1, H), lambda i: (0, 0))],
    out_specs=[pl.BlockSpec((R, HALF), lambda i: (i, 0)),
               pl.BlockSpec((R, HALF), lambda i: (i, 0))],
    out_shape=[jax.ShapeDtypeStruct((NP, HALF), jnp.float32),
               jax.ShapeDtypeStruct((NP, HALF), jnp.float32)],
)


def _combine_body(aA, aB, d0, d1, rA, rB, wl, oA, oB):
    deg = jnp.maximum(d0[...][:, :1] + d1[...][:, :1], 1.0)
    agg = jnp.concatenate([aA[...], aB[...]], axis=1) / deg
    hr = jnp.concatenate([rA[...], rB[...]], axis=1)
    o = jnp.maximum(_dot(agg, wl[...]) + hr, 0.0)
    oA[...] = o[:, :HALF]
    oB[...] = o[:, HALF:]


_combine = pl.pallas_call(
    _combine_body,
    grid=(GRID,),
    in_specs=[pl.BlockSpec((R, HALF), lambda i: (i, 0)),
              pl.BlockSpec((R, HALF), lambda i: (i, 0)),
              pl.BlockSpec((R, HALF), lambda i: (i, 0)),
              pl.BlockSpec((R, HALF), lambda i: (i, 0)),
              pl.BlockSpec((R, HALF), lambda i: (i, 0)),
              pl.BlockSpec((R, HALF), lambda i: (i, 0)),
              pl.BlockSpec((H, H), lambda i: (0, 0))],
    out_specs=[pl.BlockSpec((R, HALF), lambda i: (i, 0)),
               pl.BlockSpec((R, HALF), lambda i: (i, 0))],
    out_shape=[jax.ShapeDtypeStruct((NP, HALF), jnp.float32),
               jax.ShapeDtypeStruct((NP, HALF), jnp.float32)],
)


def _pool_body(b3, hA, hB, g_ref, acc, cnt):
    i = pl.program_id(0)

    @pl.when(i == 0)
    def _():
        acc[...] = jnp.zeros_like(acc)
        cnt[...] = jnp.zeros_like(cnt)

    brow = b3[0, 0, :]
    ids = lax.broadcasted_iota(jnp.int32, (B, R), 0)
    mask = (brow[None, :] == ids).astype(jnp.float32)
    h = jnp.concatenate([hA[...], hB[...]], axis=1)
    acc[...] += _dot(mask, h)
    cnt[...] += jnp.sum(mask, axis=1, keepdims=True)

    @pl.when(i == GRID - 1)
    def _():
        g_ref[...] = acc[...] / jnp.maximum(cnt[...][:, :1], 1.0)


_pool = pl.pallas_call(
    _pool_body,
    grid=(GRID,),
    in_specs=[pl.BlockSpec((1, 1, R), lambda i: (i, 0, 0)),
              pl.BlockSpec((R, HALF), lambda i: (i, 0)),
              pl.BlockSpec((R, HALF), lambda i: (i, 0))],
    out_specs=pl.BlockSpec((B, H), lambda i: (0, 0)),
    out_shape=jax.ShapeDtypeStruct((B, H), jnp.float32),
    scratch_shapes=[pltpu.VMEM((B, H), jnp.float32),
                    pltpu.VMEM((B, 128), jnp.float32)],
)


def _head_body(g, ts, wt1, bt1, wt2, bt2, wh1, bh1, wh2, bh2, wh3, bh3,
               wh4, bh4, out):
    t = jnp.maximum(_dot(ts[...], wt1[...]) + bt1[...], 0.0)
    t = _dot(t, wt2[...]) + bt2[...]
    fused = jnp.concatenate([g[...], t], axis=1)
    h1 = jnp.maximum(_dot(fused, wh1[...]) + bh1[...], 0.0)
    h2 = jnp.maximum(_dot(h1, wh2[...]) + bh2[...], 0.0)
    h3 = jnp.maximum(_dot(h2, wh3[...]) + bh3[...], 0.0)
    out[...] = _dot(h3, wh4[...]) + bh4[...]


_head = pl.pallas_call(
    _head_body,
    grid=(1,),
    in_specs=[pl.BlockSpec((B, H), lambda i: (0, 0)),
              pl.BlockSpec((B, T), lambda i: (0, 0)),
              pl.BlockSpec((T, H), lambda i: (0, 0)),
              pl.BlockSpec((1, H), lambda i: (0, 0)),
              pl.BlockSpec((H, H), lambda i: (0, 0)),
              pl.BlockSpec((1, H), lambda i: (0, 0)),
              pl.BlockSpec((2 * H, 512), lambda i: (0, 0)),
              pl.BlockSpec((1, 512), lambda i: (0, 0)),
              pl.BlockSpec((512, 256), lambda i: (0, 0)),
              pl.BlockSpec((1, 256), lambda i: (0, 0)),
              pl.BlockSpec((256, 128), lambda i: (0, 0)),
              pl.BlockSpec((1, 128), lambda i: (0, 0)),
              pl.BlockSpec((128, 1), lambda i: (0, 0)),
              pl.BlockSpec((1, 1), lambda i: (0, 0))],
    out_specs=pl.BlockSpec((B, 1), lambda i: (0, 0)),
    out_shape=jax.ShapeDtypeStruct((B, 1), jnp.float32),
)


def kernel(x, edge_index, batch, transform_seq, Wp, bp, conv_Wl, conv_bl,
           conv_Wr, Wt1, bt1, Wt2, bt2, Wh1, bh1, Wh2, bh2, Wh3, bh3,
           Wh4, bh4):
    src = edge_index[0]
    dst = edge_index[1]
    pad = E_PAD - E
    src_p = jnp.concatenate([src, jnp.zeros((pad,), jnp.int32)])
    dst_p = jnp.concatenate([dst, jnp.full((pad,), N, jnp.int32)])
    src3 = src_p.reshape(NS, NBLK, K)
    dst3 = dst_p.reshape(NS, NBLK, K)
    zrows = jnp.zeros((ROWS_PER, HALF), jnp.float32)
    onesk = jnp.ones((K, HALF), jnp.float32)
    dstd = dst_p.reshape(NC * NS, NBLK_D, K)
    x_p = jnp.concatenate([x, jnp.zeros((NP - N, DIN), jnp.float32)])
    batch3 = jnp.concatenate(
        [batch, jnp.full((NP - N,), B, jnp.int32)]).reshape(GRID, 1, R)

    deg0, deg1 = _deg_sc(dstd, onesk, zrows)
    hA, hB = _proj(x_p, Wp, bp.reshape(1, H))
    hA = hA + 0.0 * deg0
    for l in range(L):
        hrA, hrB = _linear(hA, hB, conv_Wr[l], conv_bl[l].reshape(1, H))
        aggA, aggB = _seg_sum_sc(hA, hB, src3, dst3, zrows)
        hA, hB = _combine(aggA, aggB, deg0, deg1, hrA, hrB, conv_Wl[l])
    g = _pool(batch3, hA, hB)
    out = _head(g, transform_seq, Wt1, bt1.reshape(1, H), Wt2,
                bt2.reshape(1, H), Wh1, bh1.reshape(1, 512), Wh2,
                bh2.reshape(1, 256), Wh3, bh3.reshape(1, 128), Wh4,
                bh4.reshape(1, 1))
    return out
